# Initial kernel scaffold; baseline (speedup 1.0000x reference)
import jax
import jax.numpy as jnp
from jax.experimental import pallas as pl


def kernel(token_positions, embeddings):
    raise NotImplementedError("write your pallas kernel here")


# speedup vs baseline: 1.9522x; 1.9522x over previous
import functools

import jax
import jax.numpy as jnp
from jax import lax
from jax.experimental import pallas as pl
from jax.experimental.pallas import tpu as pltpu
from jax.experimental.pallas import tpu_sc as plsc

_D = 1024
_B = 16384
_NW = 32
_BPW = _B // _NW
_C = 64
_NCHUNK = _BPW // _C


def _gather_body(idx_hbm, table_hbm, out_hbm, idx_v, buf, sem):
    wid = lax.axis_index("s") * 2 + lax.axis_index("c")
    pltpu.sync_copy(idx_hbm.at[wid], idx_v)
    base = wid * _BPW
    for j in range(_NCHUNK):
        pltpu.async_copy(table_hbm.at[idx_v.at[j]], buf, sem).wait()
        pltpu.sync_copy(buf, out_hbm.at[pl.ds(base + j * _C, _C)])


@jax.jit
def _run(idx, table):
    mesh = plsc.VectorSubcoreMesh(core_axis_name="c", subcore_axis_name="s")
    f = pl.kernel(
        _gather_body,
        out_type=jax.ShapeDtypeStruct((_B, _D), jnp.float32),
        mesh=mesh,
        scratch_types=[
            pltpu.VMEM((_NCHUNK, _C), jnp.int32),
            pltpu.VMEM((_C, _D), jnp.float32),
            pltpu.SemaphoreType.DMA,
        ],
    )
    return f(idx, table)


def kernel(token_positions, embeddings):
    idx = token_positions.astype(jnp.int32).reshape(_NW, _NCHUNK, _C)
    out = _run(idx, embeddings)
    return out.reshape(token_positions.shape + (_D,))


# speedup vs baseline: 1.9936x; 1.0212x over previous
import functools

import jax
import jax.numpy as jnp
from jax import lax
from jax.experimental import pallas as pl
from jax.experimental.pallas import tpu as pltpu
from jax.experimental.pallas import tpu_sc as plsc

_D = 1024
_B = 16384
_NW = 32
_BPW = _B // _NW
_C = 32
_NCHUNK = _BPW // _C


def _gather_body(idx_hbm, table_hbm, out_hbm, idx_v, buf0, buf1, g0, g1, w0, w1):
    wid = lax.axis_index("s") * 2 + lax.axis_index("c")
    pltpu.sync_copy(idx_hbm.at[wid], idx_v)
    base = wid * _BPW

    def start_gather(j, buf, sem):
        pltpu.async_copy(table_hbm.at[idx_v.at[j]], buf, sem)

    def wait_gather(buf, sem):
        pltpu.make_async_copy(table_hbm.at[idx_v.at[0]], buf, sem).wait()

    def start_write(j, buf, sem):
        pltpu.async_copy(buf, out_hbm.at[pl.ds(base + j * _C, _C)], sem)

    def wait_write(buf, sem):
        pltpu.make_async_copy(buf, out_hbm.at[pl.ds(base, _C)], sem).wait()

    start_gather(0, buf0, g0)
    start_gather(1, buf1, g1)
    wait_gather(buf0, g0)
    start_write(0, buf0, w0)

    def steady(g, _):
        j = 2 * g
        wait_gather(buf1, g1)
        start_write(j + 1, buf1, w1)
        wait_write(buf0, w0)
        start_gather(j + 2, buf0, g0)
        wait_write(buf1, w1)
        start_gather(j + 3, buf1, g1)
        wait_gather(buf0, g0)
        start_write(j + 2, buf0, w0)
        return _

    lax.fori_loop(0, _NCHUNK // 2 - 1, steady, None)

    wait_gather(buf1, g1)
    start_write(_NCHUNK - 1, buf1, w1)
    wait_write(buf0, w0)
    wait_write(buf1, w1)


@jax.jit
def _run(idx, table):
    mesh = plsc.VectorSubcoreMesh(core_axis_name="c", subcore_axis_name="s")
    f = pl.kernel(
        _gather_body,
        out_type=jax.ShapeDtypeStruct((_B, _D), jnp.float32),
        mesh=mesh,
        scratch_types=[
            pltpu.VMEM((_NCHUNK, _C), jnp.int32),
            pltpu.VMEM((_C, _D), jnp.float32),
            pltpu.VMEM((_C, _D), jnp.float32),
            pltpu.SemaphoreType.DMA,
            pltpu.SemaphoreType.DMA,
            pltpu.SemaphoreType.DMA,
            pltpu.SemaphoreType.DMA,
        ],
    )
    return f(idx, table)


def kernel(token_positions, embeddings):
    idx = token_positions.astype(jnp.int32).reshape(_NW, _NCHUNK, _C)
    out = _run(idx, embeddings)
    return out.reshape(token_positions.shape + (_D,))


# speedup vs baseline: 1.9957x; 1.0010x over previous
import jax
import jax.numpy as jnp
from jax import lax
from jax.experimental import pallas as pl
from jax.experimental.pallas import tpu as pltpu
from jax.experimental.pallas import tpu_sc as plsc

_D = 1024
_SEQ = 4096
_NW = 32
_BPW = 512
_WPR = _SEQ // _BPW
_C = 32
_NCHUNK = _BPW // _C


def _gather_body(idx_hbm, table_hbm, out_hbm, idx_v, buf0, buf1, g0, g1, w0, w1):
    wid = lax.axis_index("s") * 2 + lax.axis_index("c")
    r = wid // _WPR
    c0 = (wid % _WPR) * _BPW
    pltpu.sync_copy(idx_hbm.at[r, pl.ds(c0, _BPW)], idx_v)

    def start_gather(j, buf, sem):
        pltpu.async_copy(table_hbm.at[idx_v.at[pl.ds(j * _C, _C)]], buf, sem)

    def wait_gather(buf, sem):
        pltpu.make_async_copy(table_hbm.at[idx_v.at[pl.ds(0, _C)]], buf, sem).wait()

    def start_write(j, buf, sem):
        pltpu.async_copy(buf, out_hbm.at[r, pl.ds(c0 + j * _C, _C)], sem)

    def wait_write(buf, sem):
        pltpu.make_async_copy(buf, out_hbm.at[r, pl.ds(c0, _C)], sem).wait()

    start_gather(0, buf0, g0)
    start_gather(1, buf1, g1)
    wait_gather(buf0, g0)
    start_write(0, buf0, w0)

    def steady(g, _):
        j = 2 * g
        wait_gather(buf1, g1)
        start_write(j + 1, buf1, w1)
        wait_write(buf0, w0)
        start_gather(j + 2, buf0, g0)
        wait_write(buf1, w1)
        start_gather(j + 3, buf1, g1)
        wait_gather(buf0, g0)
        start_write(j + 2, buf0, w0)
        return _

    lax.fori_loop(0, _NCHUNK // 2 - 1, steady, None)

    wait_gather(buf1, g1)
    start_write(_NCHUNK - 1, buf1, w1)
    wait_write(buf0, w0)
    wait_write(buf1, w1)


def kernel(token_positions, embeddings):
    mesh = plsc.VectorSubcoreMesh(core_axis_name="c", subcore_axis_name="s")
    f = pl.kernel(
        _gather_body,
        out_type=jax.ShapeDtypeStruct((4, _SEQ, _D), jnp.float32),
        mesh=mesh,
        scratch_types=[
            pltpu.VMEM((_BPW,), jnp.int32),
            pltpu.VMEM((_C, _D), jnp.float32),
            pltpu.VMEM((_C, _D), jnp.float32),
            pltpu.SemaphoreType.DMA,
            pltpu.SemaphoreType.DMA,
            pltpu.SemaphoreType.DMA,
            pltpu.SemaphoreType.DMA,
        ],
    )
    return f(token_positions, embeddings)
